# baseline (speedup 1.0000x reference)
import jax
import jax.numpy as jnp
from jax import lax
from jax.experimental import pallas as pl
from jax.experimental.pallas import tpu as pltpu
from jax.experimental.pallas import tpu_sc as plsc

N = 2097152
G = 128
V = G * G * G
NC = 2
NS = 16
NW = NC * NS
PER_W = N // NW
C = 512
CR = 512
L = 16

_params = pltpu.CompilerParams(
    needs_layout_passes=False, use_tc_tiling_on_sc=False)


def _splat_i32(v):
    return jnp.full((L,), v, dtype=jnp.int32)


def _dim_index_frac(u):
    u = jnp.clip(u, 0.0, 1.0 - 1e-07)
    x = u * jnp.float32(G - 1)
    i0 = x.astype(jnp.int32)
    w1 = x - i0.astype(jnp.float32)
    return i0, w1


def _body(coords_hbm, theta_hbm, out_hbm,
          packed_hbm, in_v, pout_v, coords_v, idx_v, frac_v, rows_v, out_v,
          sem):
    cid = lax.axis_index("c")
    sid = lax.axis_index("s")
    wid = sid * NC + cid
    iota = lax.iota(jnp.int32, L)
    iota3 = iota * 3

    rows_per_tile = V // NS

    def pack_chunk(t, _):
        rbase = sid * rows_per_tile + t * CR
        pltpu.sync_copy(theta_hbm.at[pl.ds(rbase * 3, CR * 3)],
                        in_v.at[pl.ds(0, CR * 3)])

        def group(g, _):
            vv = iota + g * L
            v3 = vv * 3
            zeros = jnp.zeros((L,), jnp.float32)
            for c in range(6):
                val = plsc.load_gather(in_v, [v3 + c])
                plsc.store_scatter(pout_v, [vv, _splat_i32(c)], val)
            plsc.store_scatter(pout_v, [vv, _splat_i32(6)], zeros)
            plsc.store_scatter(pout_v, [vv, _splat_i32(7)], zeros)
            return 0

        lax.fori_loop(0, CR // L, group, 0)
        pltpu.sync_copy(pout_v, packed_hbm.at[pl.ds(rbase, CR)])
        return 0

    lax.fori_loop(0, rows_per_tile // CR, pack_chunk, 0)
    plsc.subcore_barrier()

    def chunk_body(t, _):
        base = wid * PER_W + t * C
        pltpu.sync_copy(coords_hbm.at[pl.ds(base * 3, C * 3)], coords_v)

        def make_pass1(s):
            def pass1(g, _):
                gg = s * (128 // L) + g
                off = iota3 + gg * (3 * L)
                x = plsc.load_gather(coords_v, [off])
                y = plsc.load_gather(coords_v, [off + 1])
                z = plsc.load_gather(coords_v, [off + 2])
                i0, wx1 = _dim_index_frac(x)
                j0, wy1 = _dim_index_frac(y)
                k0, wz1 = _dim_index_frac(z)
                a = i0 * (G * G) + j0 * G + k0
                b = a + G
                c = a + (G * G)
                d = c + G
                sl = pl.ds(g * L, L)
                idx_v[0, s, sl] = a
                idx_v[1, s, sl] = c
                idx_v[2, s, sl] = b
                idx_v[3, s, sl] = d
                gsl = pl.ds(gg * L, L)
                frac_v[0, gsl] = wx1
                frac_v[1, gsl] = wy1
                frac_v[2, gsl] = wz1
                return 0
            return pass1

        for s in range(C // 128):
            lax.fori_loop(0, 128 // L, make_pass1(s), 0)

        cps = [
            pltpu.async_copy(packed_hbm.at[idx_v.at[n, s]],
                             rows_v.at[n, pl.ds(s * 128, 128)], sem)
            for n in range(4)
            for s in range(C // 128)
        ]
        for cp in cps:
            cp.wait()

        def pass2(g, _):
            gsl = pl.ds(g * L, L)
            pid = iota + g * L
            wx1 = frac_v[0, gsl]
            wy1 = frac_v[1, gsl]
            wz1 = frac_v[2, gsl]
            wx0 = 1.0 - wx1
            wy0 = 1.0 - wy1
            wz0 = 1.0 - wz1
            wxy = (wx0 * wy0, wx1 * wy0, wx0 * wy1, wx1 * wy1)
            w = tuple(wz0 * t for t in wxy) + tuple(wz1 * t for t in wxy)
            p3 = iota3 + g * (3 * L)
            for ch in range(3):
                acc = w[0] * plsc.load_gather(
                    rows_v, [_splat_i32(0), pid, _splat_i32(ch)])
                for n in range(1, 8):
                    acc = acc + w[n] * plsc.load_gather(
                        rows_v,
                        [_splat_i32(n % 4), pid, _splat_i32(ch + 3 * (n // 4))])
                plsc.store_scatter(out_v, [p3 + ch], acc)
            return 0

        lax.fori_loop(0, C // L, pass2, 0)
        pltpu.sync_copy(out_v, out_hbm.at[pl.ds(base * 3, C * 3)])
        return 0

    lax.fori_loop(0, PER_W // C, chunk_body, 0)


@jax.jit
def kernel(coords, theta):
    theta_flat = theta.reshape(V * 3)
    coords_flat = coords.reshape(N * 3)
    mesh = plsc.VectorSubcoreMesh(core_axis_name="c", subcore_axis_name="s")

    run = pl.kernel(
        _body,
        out_type=jax.ShapeDtypeStruct((N * 3,), jnp.float32),
        mesh=mesh,
        scratch_types=[
            pltpu.HBM((V, 8), jnp.float32),
            pltpu.VMEM((CR * 3 + 8,), jnp.float32),
            pltpu.VMEM((CR, 8), jnp.float32),
            pltpu.VMEM((C * 3,), jnp.float32),
            pltpu.VMEM((4, C // 128, 128), jnp.int32),
            pltpu.VMEM((3, C), jnp.float32),
            pltpu.VMEM((4, C, 8), jnp.float32),
            pltpu.VMEM((C * 3,), jnp.float32),
            pltpu.SemaphoreType.DMA,
        ],
        compiler_params=_params,
    )
    return run(coords_flat, theta_flat).reshape(N, 3)


# speedup vs baseline: 3.8276x; 3.8276x over previous
import jax
import jax.numpy as jnp
from jax import lax
from jax.experimental import pallas as pl
from jax.experimental.pallas import tpu as pltpu
from jax.experimental.pallas import tpu_sc as plsc

N = 2097152
G = 128
V = G * G * G
NB = N // G
NC = 2
NS = 16
NW = NC * NS
PER_W = N // NW
C = 512
CJ = 4
CR = CJ * G
L = 16

_params = pltpu.CompilerParams(
    needs_layout_passes=False, use_tc_tiling_on_sc=False)


def _splat_i32(v):
    return jnp.full((L,), v, dtype=jnp.int32)


def _dim_index_frac(u):
    u = jnp.clip(u, 0.0, 1.0 - 1e-07)
    x = u * jnp.float32(G - 1)
    i0 = x.astype(jnp.int32)
    w1 = x - i0.astype(jnp.float32)
    return i0, w1


def _body(xyz_hbm, theta_hbm, out_hbm,
          packed_hbm, inc_v, pout_v, crd_v, idx_v, frac_v, rows_v, out_v,
          sem):
    cid = lax.axis_index("c")
    sid = lax.axis_index("s")
    wid = sid * NC + cid
    iota = lax.iota(jnp.int32, L)

    rows_per_tile = V // NS
    chunks_per_tile = rows_per_tile // CR

    def pack_chunk(t, _):
        vbase = sid * rows_per_tile + t * CR
        i_slab = vbase // (G * G)
        jk = vbase - i_slab * (G * G)
        for c in range(3):
            pltpu.sync_copy(
                theta_hbm.at[pl.ds((i_slab * 3 + c) * (G * G) + jk, CR)],
                inc_v.at[c, pl.ds(0, CR)])

        def group(g, _):
            r0 = g * L
            zeros = jnp.zeros((L,), jnp.float32)
            vv = iota + r0
            vv1 = vv + 1
            for c in range(3):
                k0v = inc_v[c, pl.ds(r0, L)]
                k1v = plsc.load_gather(inc_v, [_splat_i32(c), vv1])
                plsc.store_scatter(pout_v, [vv, _splat_i32(c)], k0v)
                plsc.store_scatter(pout_v, [vv, _splat_i32(c + 3)], k1v)
            plsc.store_scatter(pout_v, [vv, _splat_i32(6)], zeros)
            plsc.store_scatter(pout_v, [vv, _splat_i32(7)], zeros)
            return 0

        lax.fori_loop(0, CR // L, group, 0)
        pltpu.sync_copy(pout_v, packed_hbm.at[pl.ds(vbase, CR)])
        return 0

    lax.fori_loop(0, chunks_per_tile, pack_chunk, 0)
    plsc.subcore_barrier()

    def chunk_body(t, _):
        base = wid * PER_W + t * C
        for d in range(3):
            pltpu.sync_copy(xyz_hbm.at[d, pl.ds(base, C)],
                            crd_v.at[d, pl.ds(0, C)])

        def make_pass1(s):
            def pass1(g, _):
                p0 = s * G + g * L
                sl16 = pl.ds(p0, L)
                x = crd_v[0, sl16]
                y = crd_v[1, sl16]
                z = crd_v[2, sl16]
                i0, wx1 = _dim_index_frac(x)
                j0, wy1 = _dim_index_frac(y)
                k0, wz1 = _dim_index_frac(z)
                a = i0 * (G * G) + j0 * G + k0
                b = a + G
                c = a + (G * G)
                d = c + G
                sl = pl.ds(g * L, L)
                idx_v[0, s, sl] = a
                idx_v[1, s, sl] = c
                idx_v[2, s, sl] = b
                idx_v[3, s, sl] = d
                frac_v[0, sl16] = wx1
                frac_v[1, sl16] = wy1
                frac_v[2, sl16] = wz1
                return 0
            return pass1

        for s in range(C // G):
            lax.fori_loop(0, G // L, make_pass1(s), 0)

        cps = [
            pltpu.async_copy(packed_hbm.at[idx_v.at[n, s]],
                             rows_v.at[n, pl.ds(s * G, G)], sem)
            for n in range(4)
            for s in range(C // G)
        ]
        for cp in cps:
            cp.wait()

        def make_pass2(s):
            def pass2(g, _):
                p0 = s * G + g * L
                sl16 = pl.ds(p0, L)
                pid = iota + p0
                wx1 = frac_v[0, sl16]
                wy1 = frac_v[1, sl16]
                wz1 = frac_v[2, sl16]
                wx0 = 1.0 - wx1
                wy0 = 1.0 - wy1
                wz0 = 1.0 - wz1
                wxy = (wx0 * wy0, wx1 * wy0, wx0 * wy1, wx1 * wy1)
                w = tuple(wz0 * q for q in wxy) + tuple(wz1 * q for q in wxy)
                for ch in range(3):
                    acc = w[0] * plsc.load_gather(
                        rows_v, [_splat_i32(0), pid, _splat_i32(ch)])
                    for n in range(1, 8):
                        acc = acc + w[n] * plsc.load_gather(
                            rows_v, [_splat_i32(n % 4), pid,
                                     _splat_i32(ch + 3 * (n // 4))])
                    out_v[s, pl.ds(ch * G + g * L, L)] = acc
                return 0
            return pass2

        for s in range(C // G):
            lax.fori_loop(0, G // L, make_pass2(s), 0)
            zeros = jnp.zeros((L,), jnp.float32)

            def padrow(g, _):
                out_v[s, pl.ds(3 * G + g * L, L)] = zeros
                return 0

            lax.fori_loop(0, G // L, padrow, 0)

        pltpu.sync_copy(out_v, out_hbm.at[pl.ds(base // G, C // G)])
        return 0

    lax.fori_loop(0, PER_W // C, chunk_body, 0)


@jax.jit
def kernel(coords, theta):
    theta_planar = jnp.transpose(theta, (0, 3, 1, 2)).reshape(V * 3)
    xyz = jnp.transpose(coords)
    mesh = plsc.VectorSubcoreMesh(core_axis_name="c", subcore_axis_name="s")

    run = pl.kernel(
        _body,
        out_type=jax.ShapeDtypeStruct((NB, 4 * G), jnp.float32),
        mesh=mesh,
        scratch_types=[
            pltpu.HBM((V, 8), jnp.float32),
            pltpu.VMEM((3, CR + 8), jnp.float32),
            pltpu.VMEM((CR, 8), jnp.float32),
            pltpu.VMEM((3, C), jnp.float32),
            pltpu.VMEM((4, C // G, G), jnp.int32),
            pltpu.VMEM((3, C), jnp.float32),
            pltpu.VMEM((4, C, 8), jnp.float32),
            pltpu.VMEM((C // G, 4 * G), jnp.float32),
            pltpu.SemaphoreType.DMA,
        ],
        compiler_params=_params,
    )
    blocks = run(xyz, theta_planar)
    out = blocks.reshape(NB, 4, G)[:, :3, :].transpose(0, 2, 1).reshape(N, 3)
    return out


# speedup vs baseline: 7.5914x; 1.9833x over previous
import jax
import jax.numpy as jnp
from jax import lax
from jax.experimental import pallas as pl
from jax.experimental.pallas import tpu as pltpu
from jax.experimental.pallas import tpu_sc as plsc

N = 2097152
G = 128
V = G * G * G
NB = N // G
NC = 2
NS = 16
NW = NC * NS
PER_W = N // NW
C = 512
T = PER_W // C
CJ = 4
CR = CJ * G
L = 16

_params = pltpu.CompilerParams(
    needs_layout_passes=False, use_tc_tiling_on_sc=False)


def _splat_i32(v):
    return jnp.full((L,), v, dtype=jnp.int32)


def _dim_index_frac(u):
    u = jnp.clip(u, 0.0, 1.0 - 1e-07)
    x = u * jnp.float32(G - 1)
    i0 = x.astype(jnp.int32)
    w1 = x - i0.astype(jnp.float32)
    return i0, w1


def _body(xyz_hbm, theta_hbm, out_hbm, packed_hbm,
          inc_v, pout_v, crd_v, idx_v, frac_v, rows_v, out_v,
          pin0, pin1, pps0, pps1, scrd0, scrd1, sg0, sg1, sout0, sout1):
    cid = lax.axis_index("c")
    sid = lax.axis_index("s")
    wid = sid * NC + cid
    iota = lax.iota(jnp.int32, L)
    pin = (pin0, pin1)
    pps = (pps0, pps1)
    scrd = (scrd0, scrd1)
    sg = (sg0, sg1)
    sout = (sout0, sout1)

    rows_per_tile = V // NS
    PT = rows_per_tile // CR
    plane = G * G

    def pk_src(t, c):
        vbase = sid * rows_per_tile + t * CR
        vbase = jnp.minimum(vbase, sid * rows_per_tile + rows_per_tile - CR)
        i_slab = vbase // plane
        jk = vbase - i_slab * plane
        return theta_hbm.at[pl.ds((i_slab * 3 + c) * plane + jk, CR)]

    def pk_issue_in(b, t):
        for c in range(3):
            pltpu.make_async_copy(
                pk_src(t, c), inc_v.at[b, c, pl.ds(0, CR)], pin[b]).start()

    def pk_wait_in(b):
        for c in range(3):
            pltpu.make_async_copy(
                pk_src(0, c), inc_v.at[b, c, pl.ds(0, CR)], pin[b]).wait()

    def pk_compute(b):
        def group(g, _):
            r0 = g * L
            zeros = jnp.zeros((L,), jnp.float32)
            vv = iota + r0
            vv1 = vv + 1
            for c in range(3):
                k0v = inc_v[b, c, pl.ds(r0, L)]
                k1v = plsc.load_gather(inc_v, [_splat_i32(b), _splat_i32(c),
                                               vv1])
                plsc.store_scatter(pout_v, [_splat_i32(b), vv, _splat_i32(c)],
                                   k0v)
                plsc.store_scatter(pout_v,
                                   [_splat_i32(b), vv, _splat_i32(c + 3)], k1v)
            plsc.store_scatter(pout_v, [_splat_i32(b), vv, _splat_i32(6)],
                               zeros)
            plsc.store_scatter(pout_v, [_splat_i32(b), vv, _splat_i32(7)],
                               zeros)
            return 0

        lax.fori_loop(0, CR // L, group, 0)

    def pk_dst(t):
        vbase = sid * rows_per_tile + t * CR
        return packed_hbm.at[pl.ds(vbase, CR)]

    def pk_issue_out(b, t):
        pltpu.make_async_copy(pout_v.at[b], pk_dst(t), pps[b]).start()

    def pk_wait_out(b):
        pltpu.make_async_copy(pout_v.at[b], pk_dst(0), pps[b]).wait()

    pk_issue_in(0, 0)
    pk_issue_in(1, 1)

    def pack_pair(tt, _):
        for b in (0, 1):
            t = 2 * tt + b
            pk_wait_in(b)

            @pl.when(tt > 0)
            def _():
                pk_wait_out(b)

            pk_compute(b)
            pk_issue_out(b, t)

            @pl.when(tt < PT // 2 - 1)
            def _():
                pk_issue_in(b, t + 2)

        return 0

    lax.fori_loop(0, PT // 2, pack_pair, 0)
    pk_wait_out(0)
    pk_wait_out(1)
    plsc.subcore_barrier()

    def crd_src(t, d):
        base = jnp.minimum(wid * PER_W + t * C, N - C)
        return xyz_hbm.at[d, pl.ds(base, C)]

    def issue_crd(b, t):
        for d in range(3):
            pltpu.make_async_copy(
                crd_src(t, d), crd_v.at[b, d, pl.ds(0, C)], scrd[b]).start()

    def wait_crd(b):
        for d in range(3):
            pltpu.make_async_copy(
                crd_src(0, d), crd_v.at[b, d, pl.ds(0, C)], scrd[b]).wait()

    def do_pass1(b):
        def make_pass1(s):
            def pass1(g, _):
                p0 = s * G + g * L
                sl16 = pl.ds(p0, L)
                x = crd_v[b, 0, sl16]
                y = crd_v[b, 1, sl16]
                z = crd_v[b, 2, sl16]
                i0, wx1 = _dim_index_frac(x)
                j0, wy1 = _dim_index_frac(y)
                k0, wz1 = _dim_index_frac(z)
                a = i0 * plane + j0 * G + k0
                bb = a + G
                cc = a + plane
                dd = cc + G
                sl = pl.ds(g * L, L)
                idx_v[b, 0, s, sl] = a
                idx_v[b, 1, s, sl] = cc
                idx_v[b, 2, s, sl] = bb
                idx_v[b, 3, s, sl] = dd
                frac_v[b, 0, sl16] = wx1
                frac_v[b, 1, sl16] = wy1
                frac_v[b, 2, sl16] = wz1
                return 0
            return pass1

        for s in range(C // G):
            lax.fori_loop(0, G // L, make_pass1(s), 0)

    def issue_gathers(b):
        for n in range(4):
            for s in range(C // G):
                pltpu.make_async_copy(
                    packed_hbm.at[idx_v.at[b, n, s]],
                    rows_v.at[b, n, pl.ds(s * G, G)], sg[b]).start()

    def wait_gathers(b):
        for n in range(4):
            for s in range(C // G):
                pltpu.make_async_copy(
                    packed_hbm.at[idx_v.at[b, n, s]],
                    rows_v.at[b, n, pl.ds(s * G, G)], sg[b]).wait()

    def do_pass2(b):
        def make_pass2(s):
            def pass2(g, _):
                p0 = s * G + g * L
                sl16 = pl.ds(p0, L)
                pid = iota + p0
                wx1 = frac_v[b, 0, sl16]
                wy1 = frac_v[b, 1, sl16]
                wz1 = frac_v[b, 2, sl16]
                wx0 = 1.0 - wx1
                wy0 = 1.0 - wy1
                wz0 = 1.0 - wz1
                wxy = (wx0 * wy0, wx1 * wy0, wx0 * wy1, wx1 * wy1)
                w = tuple(wz0 * q for q in wxy) + tuple(wz1 * q for q in wxy)
                bsp = _splat_i32(b)
                for ch in range(3):
                    acc = w[0] * plsc.load_gather(
                        rows_v, [bsp, _splat_i32(0), pid, _splat_i32(ch)])
                    for n in range(1, 8):
                        acc = acc + w[n] * plsc.load_gather(
                            rows_v, [bsp, _splat_i32(n % 4), pid,
                                     _splat_i32(ch + 3 * (n // 4))])
                    out_v[b, s, pl.ds(ch * G + g * L, L)] = acc
                return 0
            return pass2

        for s in range(C // G):
            lax.fori_loop(0, G // L, make_pass2(s), 0)

    def out_dst(t):
        rowbase = wid * (PER_W // G) + t * (C // G)
        return out_hbm.at[pl.ds(rowbase, C // G)]

    def issue_out(b, t):
        pltpu.make_async_copy(out_v.at[b], out_dst(t), sout[b]).start()

    def wait_out(b):
        pltpu.make_async_copy(out_v.at[b], out_dst(0), sout[b]).wait()

    issue_crd(0, 0)
    issue_crd(1, 1)
    wait_crd(0)
    do_pass1(0)
    issue_gathers(0)

    def interp_pair(tt, _):
        for b in (0, 1):
            t = 2 * tt + b
            nb = 1 - b
            wait_crd(nb)
            do_pass1(nb)
            issue_gathers(nb)
            issue_crd(b, t + 2)

            @pl.when(tt > 0)
            def _():
                wait_out(b)

            wait_gathers(b)
            do_pass2(b)
            issue_out(b, t)
        return 0

    lax.fori_loop(0, T // 2, interp_pair, 0)
    wait_crd(1)
    wait_gathers(0)
    wait_out(0)
    wait_out(1)


@jax.jit
def kernel(coords, theta):
    theta_planar = jnp.transpose(theta, (0, 3, 1, 2)).reshape(V * 3)
    xyz = jnp.transpose(coords)
    mesh = plsc.VectorSubcoreMesh(core_axis_name="c", subcore_axis_name="s")

    run = pl.kernel(
        _body,
        out_type=jax.ShapeDtypeStruct((NB, 4 * G), jnp.float32),
        mesh=mesh,
        scratch_types=[
            pltpu.HBM((V, 8), jnp.float32),
            pltpu.VMEM((2, 3, CR + 8), jnp.float32),
            pltpu.VMEM((2, CR, 8), jnp.float32),
            pltpu.VMEM((2, 3, C), jnp.float32),
            pltpu.VMEM((2, 4, C // G, G), jnp.int32),
            pltpu.VMEM((2, 3, C), jnp.float32),
            pltpu.VMEM((2, 4, C, 8), jnp.float32),
            pltpu.VMEM((2, C // G, 4 * G), jnp.float32),
        ] + [pltpu.SemaphoreType.DMA] * 10,
        compiler_params=_params,
    )
    blocks = run(xyz, theta_planar)
    out = blocks.reshape(NB, 4, G)[:, :3, :].transpose(0, 2, 1).reshape(N, 3)
    return out


# speedup vs baseline: 9.0786x; 1.1959x over previous
import jax
import jax.numpy as jnp
from jax import lax
from jax.experimental import pallas as pl
from jax.experimental.pallas import tpu as pltpu
from jax.experimental.pallas import tpu_sc as plsc

N = 2097152
G = 128
V = G * G * G
NB = N // G
NC = 2
NS = 16
NW = NC * NS
PER_W = N // NW
C = 512
T = PER_W // C
CJ = 4
CR = CJ * G
L = 16

_params = pltpu.CompilerParams(
    needs_layout_passes=False, use_tc_tiling_on_sc=False)


def _splat_i32(v):
    return jnp.full((L,), v, dtype=jnp.int32)


def _dim_index_frac(u):
    u = jnp.clip(u, 0.0, 1.0 - 1e-07)
    x = u * jnp.float32(G - 1)
    i0 = x.astype(jnp.int32)
    w1 = x - i0.astype(jnp.float32)
    return i0, w1


_plane = G * G


def _pack_body(theta_hbm, packed_hbm, inc_v, pout_v, pin0, pin1, pps0, pps1):
    cid = lax.axis_index("c")
    sid = lax.axis_index("s")
    wid = sid * NC + cid
    iota = lax.iota(jnp.int32, L)
    pin = (pin0, pin1)
    pps = (pps0, pps1)
    rows_per_w = V // NW
    PT = rows_per_w // CR

    def pk_src(t, c):
        vbase = wid * rows_per_w + t * CR
        i_slab = vbase // _plane
        jk = vbase - i_slab * _plane
        return theta_hbm.at[pl.ds((i_slab * 3 + c) * _plane + jk, CR)]

    def pk_issue_in(b, t):
        for c in range(3):
            pltpu.make_async_copy(
                pk_src(t, c), inc_v.at[b, c, pl.ds(0, CR)], pin[b]).start()

    def pk_wait_in(b):
        for c in range(3):
            pltpu.make_async_copy(
                pk_src(0, c), inc_v.at[b, c, pl.ds(0, CR)], pin[b]).wait()

    def pk_compute(b):
        def group(g, _):
            r0 = g * L
            zeros = jnp.zeros((L,), jnp.float32)
            vv = iota + r0
            vv8 = vv * 8
            vv1 = vv + 1
            for c in range(3):
                k0v = inc_v[b, c, pl.ds(r0, L)]
                k1v = plsc.load_gather(inc_v, [_splat_i32(b), _splat_i32(c),
                                               vv1])
                plsc.store_scatter(pout_v, [_splat_i32(b), vv, _splat_i32(c)],
                                   k0v)
                plsc.store_scatter(pout_v,
                                   [_splat_i32(b), vv, _splat_i32(c + 3)], k1v)
            plsc.store_scatter(pout_v, [_splat_i32(b), vv, _splat_i32(6)],
                               zeros)
            plsc.store_scatter(pout_v, [_splat_i32(b), vv, _splat_i32(7)],
                               zeros)
            return 0

        lax.fori_loop(0, CR // L, group, 0)

    def pk_dst(t):
        vbase = wid * rows_per_w + t * CR
        return packed_hbm.at[pl.ds(vbase, CR)]

    def pk_issue_out(b, t):
        pltpu.make_async_copy(pout_v.at[b], pk_dst(t), pps[b]).start()

    def pk_wait_out(b):
        pltpu.make_async_copy(pout_v.at[b], pk_dst(0), pps[b]).wait()

    pk_issue_in(0, 0)
    pk_issue_in(1, 1)

    def pack_pair(tt, _):
        for b in (0, 1):
            t = 2 * tt + b
            pk_wait_in(b)

            @pl.when(tt > 0)
            def _():
                pk_wait_out(b)

            pk_compute(b)
            pk_issue_out(b, t)

            @pl.when(tt < PT // 2 - 1)
            def _():
                pk_issue_in(b, t + 2)

        return 0

    lax.fori_loop(0, PT // 2, pack_pair, 0)
    pk_wait_out(0)
    pk_wait_out(1)


def _interp_body(xyz_hbm, packed_full_hbm, out_hbm,
                 crd_v, idx_v, frac_v, rows_v, out_v,
                 scrd0, scrd1, sg0, sg1, sout0, sout1):
    cid = lax.axis_index("c")
    sid = lax.axis_index("s")
    wid = sid * NC + cid
    iota = lax.iota(jnp.int32, L)
    scrd = (scrd0, scrd1)
    sg = (sg0, sg1)
    sout = (sout0, sout1)
    packed_hbm = packed_full_hbm

    def crd_src(t, d):
        base = jnp.minimum(wid * PER_W + t * C, N - C)
        return xyz_hbm.at[d, pl.ds(base, C)]

    def issue_crd(b, t):
        for d in range(3):
            pltpu.make_async_copy(
                crd_src(t, d), crd_v.at[b, d, pl.ds(0, C)], scrd[b]).start()

    def wait_crd(b):
        for d in range(3):
            pltpu.make_async_copy(
                crd_src(0, d), crd_v.at[b, d, pl.ds(0, C)], scrd[b]).wait()

    def do_pass1(b):
        def make_pass1(s):
            def pass1(g, _):
                p0 = s * G + g * L
                sl16 = pl.ds(p0, L)
                x = crd_v[b, 0, sl16]
                y = crd_v[b, 1, sl16]
                z = crd_v[b, 2, sl16]
                i0, wx1 = _dim_index_frac(x)
                j0, wy1 = _dim_index_frac(y)
                k0, wz1 = _dim_index_frac(z)
                a = i0 * _plane + j0 * G + k0
                bb = a + G
                cc = a + _plane
                dd = cc + G
                sl = pl.ds(g * L, L)
                idx_v[b, 0, s, sl] = a
                idx_v[b, 1, s, sl] = cc
                idx_v[b, 2, s, sl] = bb
                idx_v[b, 3, s, sl] = dd
                frac_v[b, 0, sl16] = wx1
                frac_v[b, 1, sl16] = wy1
                frac_v[b, 2, sl16] = wz1
                return 0
            return pass1

        for s in range(C // G):
            lax.fori_loop(0, G // L, make_pass1(s), 0)

    def issue_gathers(b):
        for n in range(4):
            for s in range(C // G):
                pltpu.make_async_copy(
                    packed_hbm.at[idx_v.at[b, n, s]],
                    rows_v.at[b, n, pl.ds(s * G, G)], sg[b]).start()

    def wait_gathers(b):
        for n in range(4):
            for s in range(C // G):
                pltpu.make_async_copy(
                    packed_hbm.at[idx_v.at[b, n, s]],
                    rows_v.at[b, n, pl.ds(s * G, G)], sg[b]).wait()

    def do_pass2(b):
        def make_pass2(s):
            def pass2(g, _):
                p0 = s * G + g * L
                sl16 = pl.ds(p0, L)
                pid = iota + p0
                wx1 = frac_v[b, 0, sl16]
                wy1 = frac_v[b, 1, sl16]
                wz1 = frac_v[b, 2, sl16]
                wx0 = 1.0 - wx1
                wy0 = 1.0 - wy1
                wz0 = 1.0 - wz1
                wxy = (wx0 * wy0, wx1 * wy0, wx0 * wy1, wx1 * wy1)
                w = tuple(wz0 * q for q in wxy) + tuple(wz1 * q for q in wxy)
                bsp = _splat_i32(b)
                for ch in range(3):
                    acc = w[0] * plsc.load_gather(
                        rows_v, [bsp, _splat_i32(0), pid, _splat_i32(ch)])
                    for n in range(1, 8):
                        acc = acc + w[n] * plsc.load_gather(
                            rows_v, [bsp, _splat_i32(n % 4), pid,
                                     _splat_i32(ch + 3 * (n // 4))])
                    out_v[b, s, pl.ds(ch * G + g * L, L)] = acc
                return 0
            return pass2

        for s in range(C // G):
            lax.fori_loop(0, G // L, make_pass2(s), 0)

    def out_dst(t):
        rowbase = wid * (PER_W // G) + t * (C // G)
        return out_hbm.at[pl.ds(rowbase, C // G)]

    def issue_out(b, t):
        pltpu.make_async_copy(out_v.at[b], out_dst(t), sout[b]).start()

    def wait_out(b):
        pltpu.make_async_copy(out_v.at[b], out_dst(0), sout[b]).wait()

    issue_crd(0, 0)
    issue_crd(1, 1)
    wait_crd(0)
    do_pass1(0)
    issue_gathers(0)

    def interp_pair(tt, _):
        for b in (0, 1):
            t = 2 * tt + b
            nb = 1 - b
            wait_crd(nb)
            do_pass1(nb)
            issue_gathers(nb)
            issue_crd(b, t + 2)

            @pl.when(tt > 0)
            def _():
                wait_out(b)

            wait_gathers(b)
            do_pass2(b)
            issue_out(b, t)
        return 0

    lax.fori_loop(0, T // 2, interp_pair, 0)
    wait_crd(1)
    wait_gathers(0)
    wait_out(0)
    wait_out(1)


@jax.jit
def kernel(coords, theta):
    theta_planar = jnp.transpose(theta, (0, 3, 1, 2)).reshape(V * 3)
    xyz = jnp.transpose(coords)
    mesh = plsc.VectorSubcoreMesh(core_axis_name="c", subcore_axis_name="s")

    pack = pl.kernel(
        _pack_body,
        out_type=jax.ShapeDtypeStruct((V, 8), jnp.float32),
        mesh=mesh,
        scratch_types=[
            pltpu.VMEM((2, 3, CR + 8), jnp.float32),
            pltpu.VMEM((2, CR, 8), jnp.float32),
        ] + [pltpu.SemaphoreType.DMA] * 4,
        compiler_params=_params,
    )
    packed = pack(theta_planar)

    interp = pl.kernel(
        _interp_body,
        out_type=jax.ShapeDtypeStruct((NB, 4 * G), jnp.float32),
        mesh=mesh,
        scratch_types=[
            pltpu.VMEM((2, 3, C), jnp.float32),
            pltpu.VMEM((2, 4, C // G, G), jnp.int32),
            pltpu.VMEM((2, 3, C), jnp.float32),
            pltpu.VMEM((2, 4, C, 8), jnp.float32),
            pltpu.VMEM((2, C // G, 4 * G), jnp.float32),
        ] + [pltpu.SemaphoreType.DMA] * 6,
        compiler_params=_params,
    )
    blocks = interp(xyz, packed)
    out = blocks.reshape(NB, 4, G)[:, :3, :].transpose(0, 2, 1).reshape(N, 3)
    return out


# speedup vs baseline: 14.6401x; 1.6126x over previous
import jax
import jax.numpy as jnp
from jax import lax
from jax.experimental import pallas as pl
from jax.experimental.pallas import tpu as pltpu
from jax.experimental.pallas import tpu_sc as plsc

N = 2097152
G = 128
V = G * G * G
NB = N // G
NC = 2
NS = 16
NW = NC * NS
PER_W = N // NW
C = 1024
CB = C // G
T = PER_W // C
CJ = 4
CR = CJ * G
L = 16

_params = pltpu.CompilerParams(
    needs_layout_passes=False, use_tc_tiling_on_sc=False)


def _splat_i32(v):
    return jnp.full((L,), v, dtype=jnp.int32)


def _dim_index_frac(u):
    u = jnp.clip(u, 0.0, 1.0 - 1e-07)
    x = u * jnp.float32(G - 1)
    i0 = x.astype(jnp.int32)
    w1 = x - i0.astype(jnp.float32)
    return i0, w1


_plane = G * G


def _pack_body(theta_hbm, packed_hbm, inc_v, pout_v, pin0, pin1, pps0, pps1):
    cid = lax.axis_index("c")
    sid = lax.axis_index("s")
    wid = sid * NC + cid
    iota = lax.iota(jnp.int32, L)
    pin = (pin0, pin1)
    pps = (pps0, pps1)
    rows_per_w = V // NW
    PT = rows_per_w // CR

    def pk_src(t, c):
        vbase = wid * rows_per_w + t * CR
        i_slab = vbase // _plane
        jk = vbase - i_slab * _plane
        return theta_hbm.at[pl.ds((i_slab * 3 + c) * _plane + jk, CR)]

    def pk_issue_in(b, t):
        for c in range(3):
            pltpu.make_async_copy(
                pk_src(t, c), inc_v.at[b, c, pl.ds(0, CR)], pin[b]).start()

    def pk_wait_in(b):
        for c in range(3):
            pltpu.make_async_copy(
                pk_src(0, c), inc_v.at[b, c, pl.ds(0, CR)], pin[b]).wait()

    def pk_compute(b):
        def group(g, _):
            r0 = g * L
            zeros = jnp.zeros((L,), jnp.float32)
            vv = iota + r0
            vv8 = vv * 8
            vv1 = vv + 1
            for c in range(3):
                k0v = inc_v[b, c, pl.ds(r0, L)]
                k1v = plsc.load_gather(inc_v, [_splat_i32(b), _splat_i32(c),
                                               vv1])
                plsc.store_scatter(pout_v, [_splat_i32(b), vv, _splat_i32(c)],
                                   k0v)
                plsc.store_scatter(pout_v,
                                   [_splat_i32(b), vv, _splat_i32(c + 3)], k1v)
            plsc.store_scatter(pout_v, [_splat_i32(b), vv, _splat_i32(6)],
                               zeros)
            plsc.store_scatter(pout_v, [_splat_i32(b), vv, _splat_i32(7)],
                               zeros)
            return 0

        lax.fori_loop(0, CR // L, group, 0)

    def pk_dst(t):
        vbase = wid * rows_per_w + t * CR
        return packed_hbm.at[pl.ds(vbase, CR)]

    def pk_issue_out(b, t):
        pltpu.make_async_copy(pout_v.at[b], pk_dst(t), pps[b]).start()

    def pk_wait_out(b):
        pltpu.make_async_copy(pout_v.at[b], pk_dst(0), pps[b]).wait()

    pk_issue_in(0, 0)
    pk_issue_in(1, 1)

    def pack_pair(tt, _):
        for b in (0, 1):
            t = 2 * tt + b
            pk_wait_in(b)

            @pl.when(tt > 0)
            def _():
                pk_wait_out(b)

            pk_compute(b)
            pk_issue_out(b, t)

            @pl.when(tt < PT // 2 - 1)
            def _():
                pk_issue_in(b, t + 2)

        return 0

    lax.fori_loop(0, PT // 2, pack_pair, 0)
    pk_wait_out(0)
    pk_wait_out(1)


def _interp_body(xyz_hbm, packed_full_hbm, out_hbm,
                 crd_v, idx_v, frac_v, rows_v, out_v,
                 scrd0, scrd1, sg0, sg1, sout0, sout1):
    cid = lax.axis_index("c")
    sid = lax.axis_index("s")
    wid = sid * NC + cid
    iota = lax.iota(jnp.int32, L)
    scrd = (scrd0, scrd1)
    sg = (sg0, sg1)
    sout = (sout0, sout1)
    packed_hbm = packed_full_hbm

    def crd_src(t):
        bb = jnp.minimum((wid * PER_W + t * C) // G, NB - CB)
        return xyz_hbm.at[pl.ds(bb, CB)]

    def issue_crd(b, t):
        pltpu.make_async_copy(crd_src(t), crd_v.at[b], scrd[b]).start()

    def wait_crd(b):
        pltpu.make_async_copy(crd_src(0), crd_v.at[b], scrd[b]).wait()

    def do_pass1(b):
        def make_pass1(s):
            def pass1(g, _):
                p0 = s * G + g * L
                sl16 = pl.ds(p0, L)
                gsl = pl.ds(g * L, L)
                x = crd_v[b, s, 0, gsl]
                y = crd_v[b, s, 1, gsl]
                z = crd_v[b, s, 2, gsl]
                i0, wx1 = _dim_index_frac(x)
                j0, wy1 = _dim_index_frac(y)
                k0, wz1 = _dim_index_frac(z)
                a = i0 * _plane + j0 * G + k0
                bb = a + G
                cc = a + _plane
                dd = cc + G
                sl = pl.ds(g * L, L)
                idx_v[b, 0, s, sl] = a
                idx_v[b, 1, s, sl] = cc
                idx_v[b, 2, s, sl] = bb
                idx_v[b, 3, s, sl] = dd
                frac_v[b, 0, sl16] = wx1
                frac_v[b, 1, sl16] = wy1
                frac_v[b, 2, sl16] = wz1
                return 0
            return pass1

        for s in range(C // G):
            lax.fori_loop(0, G // L, make_pass1(s), 0)

    def issue_gathers(b):
        for n in range(4):
            for s in range(C // G):
                pltpu.make_async_copy(
                    packed_hbm.at[idx_v.at[b, n, s]],
                    rows_v.at[b, n, pl.ds(s * G, G)], sg[b]).start()

    def wait_gathers(b):
        for n in range(4):
            for s in range(C // G):
                pltpu.make_async_copy(
                    packed_hbm.at[idx_v.at[b, n, s]],
                    rows_v.at[b, n, pl.ds(s * G, G)], sg[b]).wait()

    def do_pass2(b):
        def make_pass2(s):
            def pass2(g, _):
                p0 = s * G + g * L
                sl16 = pl.ds(p0, L)
                pid = iota + p0
                wx1 = frac_v[b, 0, sl16]
                wy1 = frac_v[b, 1, sl16]
                wz1 = frac_v[b, 2, sl16]
                wx0 = 1.0 - wx1
                wy0 = 1.0 - wy1
                wz0 = 1.0 - wz1
                wxy = (wx0 * wy0, wx1 * wy0, wx0 * wy1, wx1 * wy1)
                w = tuple(wz0 * q for q in wxy) + tuple(wz1 * q for q in wxy)
                bsp = _splat_i32(b)
                for ch in range(3):
                    acc = w[0] * plsc.load_gather(
                        rows_v, [bsp, _splat_i32(0), pid, _splat_i32(ch)])
                    for n in range(1, 8):
                        acc = acc + w[n] * plsc.load_gather(
                            rows_v, [bsp, _splat_i32(n % 4), pid,
                                     _splat_i32(ch + 3 * (n // 4))])
                    out_v[b, s, pl.ds(ch * G + g * L, L)] = acc
                return 0
            return pass2

        for s in range(C // G):
            lax.fori_loop(0, G // L, make_pass2(s), 0)

    def out_dst(t):
        rowbase = wid * (PER_W // G) + t * (C // G)
        return out_hbm.at[pl.ds(rowbase, C // G)]

    def issue_out(b, t):
        pltpu.make_async_copy(out_v.at[b], out_dst(t), sout[b]).start()

    def wait_out(b):
        pltpu.make_async_copy(out_v.at[b], out_dst(0), sout[b]).wait()

    issue_crd(0, 0)
    issue_crd(1, 1)
    wait_crd(0)
    do_pass1(0)
    issue_gathers(0)

    def interp_pair(tt, _):
        for b in (0, 1):
            t = 2 * tt + b
            nb = 1 - b
            wait_crd(nb)
            do_pass1(nb)
            issue_gathers(nb)
            issue_crd(b, t + 2)

            @pl.when(tt > 0)
            def _():
                wait_out(b)

            wait_gathers(b)
            do_pass2(b)
            issue_out(b, t)
        return 0

    lax.fori_loop(0, T // 2, interp_pair, 0)
    wait_crd(1)
    wait_gathers(0)
    wait_out(0)
    wait_out(1)


@jax.jit
def kernel(coords, theta):
    theta_planar = jnp.transpose(theta, (0, 3, 1, 2)).reshape(V * 3)
    xyz = jnp.transpose(coords.reshape(NB, G, 3), (0, 2, 1))
    mesh = plsc.VectorSubcoreMesh(core_axis_name="c", subcore_axis_name="s")

    pack = pl.kernel(
        _pack_body,
        out_type=jax.ShapeDtypeStruct((V, 8), jnp.float32),
        mesh=mesh,
        scratch_types=[
            pltpu.VMEM((2, 3, CR + 8), jnp.float32),
            pltpu.VMEM((2, CR, 8), jnp.float32),
        ] + [pltpu.SemaphoreType.DMA] * 4,
        compiler_params=_params,
    )
    packed = pack(theta_planar)

    interp = pl.kernel(
        _interp_body,
        out_type=jax.ShapeDtypeStruct((NB, 4 * G), jnp.float32),
        mesh=mesh,
        scratch_types=[
            pltpu.VMEM((2, CB, 3, G), jnp.float32),
            pltpu.VMEM((2, 4, CB, G), jnp.int32),
            pltpu.VMEM((2, 3, C), jnp.float32),
            pltpu.VMEM((2, 4, C, 8), jnp.float32),
            pltpu.VMEM((2, CB, 4 * G), jnp.float32),
        ] + [pltpu.SemaphoreType.DMA] * 6,
        compiler_params=_params,
    )
    blocks = interp(xyz, packed)
    out = blocks.reshape(NB, 4, G)[:, :3, :].transpose(0, 2, 1).reshape(N, 3)
    return out
